# baseline (speedup 1.0000x reference)
import functools

import jax
import jax.numpy as jnp
from jax import lax
from jax.experimental import pallas as pl
from jax.experimental.pallas import tpu as pltpu
from jax.experimental.pallas import tpu_sc as plsc

NC, NS = 2, 16
EB = 128
LAT = 256


def _ln(h, g, be, j):
    mu_h = jnp.dot(h, j, precision=lax.Precision.HIGHEST,
                   preferred_element_type=jnp.float32)
    m2_h = jnp.dot(h * h, j, precision=lax.Precision.HIGHEST,
                   preferred_element_type=jnp.float32)
    mu = jnp.concatenate([mu_h, mu_h], axis=1)
    m2 = jnp.concatenate([m2_h, m2_h], axis=1)
    var = m2 - mu * mu
    return (h - mu) * lax.rsqrt(var + 1e-5) * g + be


def _mlp2_body(n_mm, n_add, *refs):
    xs = refs[:n_mm]
    ws = refs[n_mm:2 * n_mm]
    adds = refs[2 * n_mm:2 * n_mm + n_add]
    b1, g1, be1, w2, b2, g2, be2, j_ref, out = refs[2 * n_mm + n_add:]
    j = j_ref[...]
    def bdot(a, b):
        return jnp.dot(a.astype(jnp.bfloat16), b.astype(jnp.bfloat16),
                       preferred_element_type=jnp.float32)

    acc = bdot(xs[0][...], ws[0][...])
    for k in range(1, n_mm):
        acc = acc + bdot(xs[k][...], ws[k][...])
    for a in adds:
        acc = acc + a[...].astype(jnp.float32)
    h = _ln(jax.nn.relu(acc + b1[...]), g1[...], be1[...], j)
    h2 = jax.nn.relu(bdot(h, w2[...]) + b2[...])
    out[...] = _ln(h2, g2[...], be2[...], j)


def _mlp2(mm_inputs, add_inputs, l1, l2, m, block_rows):
    n_mm, n_add = len(mm_inputs), len(add_inputs)
    b1, g1, be1 = (p.reshape(1, LAT) for p in l1)
    w2 = l2[0]
    b2, g2, be2 = (p.reshape(1, LAT) for p in l2[1:])
    row_spec = lambda k: pl.BlockSpec((block_rows, k), lambda i: (i, 0))
    full_spec = lambda s: pl.BlockSpec(s, lambda i: (0, 0))
    in_specs = ([row_spec(x.shape[1]) for x, _ in mm_inputs]
                + [full_spec(w.shape) for _, w in mm_inputs]
                + [row_spec(LAT) for _ in add_inputs]
                + [full_spec((1, LAT))] * 3
                + [full_spec((LAT, LAT))]
                + [full_spec((1, LAT))] * 3
                + [full_spec((LAT, 128))])
    jmat = jnp.full((LAT, 128), 1.0 / LAT, jnp.float32)
    args = ([x for x, _ in mm_inputs] + [w for _, w in mm_inputs]
            + list(add_inputs) + [b1, g1, be1, w2, b2, g2, be2, jmat])
    return pl.pallas_call(
        functools.partial(_mlp2_body, n_mm, n_add),
        grid=(m // block_rows,),
        in_specs=in_specs,
        out_specs=pl.BlockSpec((block_rows, LAT), lambda i: (i, 0)),
        out_shape=jax.ShapeDtypeStruct((m, LAT), jnp.float32),
    )(*args)


def _proj_body(x_ref, wa_ref, wb_ref, oa_ref, ob_ref):
    x = x_ref[...].astype(jnp.bfloat16)
    oa_ref[...] = jnp.dot(x, wa_ref[...].astype(jnp.bfloat16),
                          preferred_element_type=jnp.float32)
    ob_ref[...] = jnp.dot(x, wb_ref[...].astype(jnp.bfloat16),
                          preferred_element_type=jnp.float32)


def _proj2(x, wa, wb, block_rows):
    m = x.shape[0]
    return pl.pallas_call(
        _proj_body,
        grid=(m // block_rows,),
        in_specs=[pl.BlockSpec((block_rows, LAT), lambda i: (i, 0)),
                  pl.BlockSpec((LAT, LAT), lambda i: (0, 0)),
                  pl.BlockSpec((LAT, LAT), lambda i: (0, 0))],
        out_specs=[pl.BlockSpec((block_rows, LAT), lambda i: (i, 0)),
                   pl.BlockSpec((block_rows, LAT), lambda i: (i, 0))],
        out_shape=[jax.ShapeDtypeStruct((m, LAT), jnp.float32),
                   jax.ShapeDtypeStruct((m, LAT), jnp.float32)],
    )(x, wa, wb)


def _sc_gather(table, idx):
    e, d = idx.shape[0], table.shape[1]
    nb = e // EB
    nw = NC * NS
    nbw = nb // nw
    mesh = plsc.VectorSubcoreMesh(core_axis_name="c", subcore_axis_name="s")

    @functools.partial(
        pl.kernel, mesh=mesh,
        out_type=jax.ShapeDtypeStruct((e, d), table.dtype),
        scratch_types=[pltpu.VMEM((EB,), jnp.int32),
                       pltpu.VMEM((EB, d), table.dtype),
                       pltpu.SemaphoreType.DMA],
    )
    def k(t_h, i_h, o_h, idx_v, rows_v, sem):
        wid = lax.axis_index("s") * NC + lax.axis_index("c")

        def one(b, carry):
            base = (b * nw + wid) * EB
            pltpu.sync_copy(i_h.at[pl.ds(base, EB)], idx_v)
            pltpu.async_copy(t_h.at[idx_v], rows_v, sem).wait()
            pltpu.sync_copy(rows_v, o_h.at[pl.ds(base, EB)])
            return carry

        lax.fori_loop(0, nbw, one, 0)

    return k(table, idx)


def _sc_segsum(e_arr, recv2, zeros, n):
    nb = recv2.shape[0]
    half = LAT // NC
    n_pad = ((n + NS * 8 - 1) // (NS * 8)) * (NS * 8)
    rows_per_tile = n_pad // NS
    nbs = nb // NS
    mesh = plsc.VectorSubcoreMesh(core_axis_name="c", subcore_axis_name="s")

    @functools.partial(
        pl.kernel, mesh=mesh,
        out_type=jax.ShapeDtypeStruct((NC, n_pad, half), jnp.float32),
        scratch_types=[pltpu.VMEM((nbs, EB), jnp.int32),
                       pltpu.VMEM((EB, half), jnp.float32),
                       pltpu.VMEM((EB, half), jnp.float32),
                       pltpu.VMEM_SHARED((n_pad, half), jnp.float32),
                       pltpu.SemaphoreType.DMA, pltpu.SemaphoreType.DMA],
    )
    def k(e_hbm, r_hbm, z_hbm, out_hbm, idxs, eb0, eb1, shared, se0, se1):
        c = lax.axis_index("c")
        s = lax.axis_index("s")
        ebuf = (eb0, eb1)
        se = (se0, se1)
        pltpu.sync_copy(z_hbm, shared.at[pl.ds(s * rows_per_tile, rows_per_tile)])
        pltpu.sync_copy(r_hbm.at[pl.ds(s * nbs, nbs)], idxs)
        plsc.subcore_barrier()

        def start_e(b, p):
            base = (s * nbs + b) * EB
            pltpu.async_copy(
                e_hbm.at[pl.ds(base, EB), pl.ds(c * half, half)],
                ebuf[p], se[p])

        def wait_e(p):
            pltpu.make_async_copy(
                e_hbm.at[pl.ds(0, EB), pl.ds(0, half)], ebuf[p],
                se[p]).wait()

        start_e(0, 0)
        start_e(1, 1)

        def pair(k2, carry):
            b0 = 2 * k2
            b1 = b0 + 1
            wait_e(0)
            pltpu.sync_copy(ebuf[0], shared.at[idxs.at[b0]], add=True)

            @pl.when(b0 + 2 < nbs)
            def _():
                start_e(b0 + 2, 0)

            wait_e(1)
            pltpu.sync_copy(ebuf[1], shared.at[idxs.at[b1]], add=True)

            @pl.when(b1 + 2 < nbs)
            def _():
                start_e(b1 + 2, 1)

            return carry

        lax.fori_loop(0, nbs // 2, pair, 0)
        plsc.subcore_barrier()
        pltpu.sync_copy(shared.at[pl.ds(s * rows_per_tile, rows_per_tile)],
                        out_hbm.at[c, pl.ds(s * rows_per_tile, rows_per_tile)])

    return k(e_arr, recv2, zeros)


def kernel(nodes, edges, senders, receivers, n_node, params):
    n, f = nodes.shape
    e_cnt = edges.shape[0]
    nw = NC * NS
    quantum = EB * nw * 2
    e_pad = ((e_cnt + quantum - 1) // quantum) * quantum
    pad = e_pad - e_cnt
    n_pad = ((n + NS * 8 - 1) // (NS * 8)) * (NS * 8)

    send_g = jnp.concatenate([senders, jnp.zeros((pad,), jnp.int32)])
    recv_g = jnp.concatenate([receivers, jnp.zeros((pad,), jnp.int32)])
    recv_s = jnp.concatenate(
        [receivers, jnp.full((pad,), n, jnp.int32)]).reshape(-1, EB)

    sp = params['sender']
    x = _mlp2([(nodes, sp[0][0])], [], sp[0][1:], sp[1], m=n, block_rows=1000)
    x = x + (jnp.asarray(n_node) - n).astype(jnp.float32)

    ep = params['edge0']
    k3 = ep[0][0].shape[0]
    edges8 = jnp.zeros((e_pad, 8), jnp.float32).at[:e_cnt, :k3].set(edges)
    w0 = jnp.concatenate([ep[0][0], jnp.zeros((8 - k3, LAT), jnp.float32)],
                         axis=0)
    e = _mlp2([(edges8, w0)], [], ep[0][1:], ep[1], m=e_pad, block_rows=640)

    zeros = jnp.zeros((n_pad // NS, LAT // NC), jnp.float32)

    for i in range(len(params['edge_steps'])):
        eps = params['edge_steps'][i]
        nps = params['node_steps'][i]
        w1 = eps[0][0]
        we, ws, wr = w1[:LAT], w1[LAT:2 * LAT], w1[2 * LAT:]
        ps, pr = _proj2(x, ws, wr, block_rows=1000)
        gs = _sc_gather(ps, send_g)
        gr = _sc_gather(pr, recv_g)
        e = _mlp2([(e, we)], [gs, gr], eps[0][1:], eps[1],
                  m=e_pad, block_rows=640)
        agg = _sc_segsum(e, recv_s, zeros, n)
        wn1 = nps[0][0]
        half = LAT // NC
        x = _mlp2([(x, wn1[:LAT]),
                   (agg[0], wn1[LAT:LAT + half]),
                   (agg[1], wn1[LAT + half:])],
                  [], nps[0][1:], nps[1], m=n, block_rows=1000)

    return (x, e[:e_cnt])


# speedup vs baseline: 1.6723x; 1.6723x over previous
import functools

import jax
import jax.numpy as jnp
from jax import lax
from jax.experimental import pallas as pl
from jax.experimental.pallas import tpu as pltpu
from jax.experimental.pallas import tpu_sc as plsc

NC, NS = 2, 16
EB = 128
LAT = 256


def _ln(h, g, be):
    mu = jnp.mean(h, axis=-1, keepdims=True)
    d = h - mu
    var = jnp.mean(d * d, axis=-1, keepdims=True)
    return d * lax.rsqrt(var + 1e-5) * g + be


def _mlp2_body(n_mm, n_add, *refs):
    xs = refs[:n_mm]
    ws = refs[n_mm:2 * n_mm]
    adds = refs[2 * n_mm:2 * n_mm + n_add]
    b1, g1, be1, w2, b2, g2, be2, out = refs[2 * n_mm + n_add:]
    acc = jnp.dot(xs[0][...], ws[0][...], preferred_element_type=jnp.float32)
    for k in range(1, n_mm):
        acc = acc + jnp.dot(xs[k][...], ws[k][...],
                            preferred_element_type=jnp.float32)
    for a in adds:
        acc = acc + a[...]
    h = _ln(jax.nn.relu(acc + b1[...]), g1[...], be1[...])
    h2 = jax.nn.relu(jnp.dot(h, w2[...], preferred_element_type=jnp.float32)
                     + b2[...])
    out[...] = _ln(h2, g2[...], be2[...])


def _mlp2(mm_inputs, add_inputs, l1, l2, block_rows):
    m = mm_inputs[0][0].shape[0]
    n_mm, n_add = len(mm_inputs), len(add_inputs)
    b1, g1, be1 = (p.reshape(1, LAT) for p in l1)
    w2 = l2[0]
    b2, g2, be2 = (p.reshape(1, LAT) for p in l2[1:])
    row_spec = lambda k: pl.BlockSpec((block_rows, k), lambda i: (i, 0))
    full_spec = lambda s: pl.BlockSpec(s, lambda i: (0, 0))
    in_specs = ([row_spec(x.shape[1]) for x, _ in mm_inputs]
                + [full_spec(w.shape) for _, w in mm_inputs]
                + [row_spec(LAT) for _ in add_inputs]
                + [full_spec((1, LAT))] * 3
                + [full_spec((LAT, LAT))]
                + [full_spec((1, LAT))] * 3)
    args = ([x for x, _ in mm_inputs] + [w for _, w in mm_inputs]
            + list(add_inputs) + [b1, g1, be1, w2, b2, g2, be2])
    return pl.pallas_call(
        functools.partial(_mlp2_body, n_mm, n_add),
        grid=(m // block_rows,),
        in_specs=in_specs,
        out_specs=pl.BlockSpec((block_rows, LAT), lambda i: (i, 0)),
        out_shape=jax.ShapeDtypeStruct((m, LAT), jnp.float32),
    )(*args)


def _proj_body(x_ref, wa_ref, wb_ref, oa_ref, ob_ref):
    x = x_ref[...]
    oa_ref[...] = jnp.dot(x, wa_ref[...], preferred_element_type=jnp.float32)
    ob_ref[...] = jnp.dot(x, wb_ref[...], preferred_element_type=jnp.float32)


def _proj2(x, wa, wb, block_rows):
    m = x.shape[0]
    return pl.pallas_call(
        _proj_body,
        grid=(m // block_rows,),
        in_specs=[pl.BlockSpec((block_rows, LAT), lambda i: (i, 0)),
                  pl.BlockSpec((LAT, LAT), lambda i: (0, 0)),
                  pl.BlockSpec((LAT, LAT), lambda i: (0, 0))],
        out_specs=[pl.BlockSpec((block_rows, LAT), lambda i: (i, 0)),
                   pl.BlockSpec((block_rows, LAT), lambda i: (i, 0))],
        out_shape=[jax.ShapeDtypeStruct((m, LAT), jnp.float32),
                   jax.ShapeDtypeStruct((m, LAT), jnp.float32)],
    )(x, wa, wb)


def _sc_gather(table, idx):
    e, d = idx.shape[0], table.shape[1]
    nb = e // EB
    nw = NC * NS
    mesh = plsc.VectorSubcoreMesh(core_axis_name="c", subcore_axis_name="s")

    @functools.partial(
        pl.kernel, mesh=mesh,
        out_type=jax.ShapeDtypeStruct((e, d), jnp.float32),
        scratch_types=[pltpu.VMEM((EB,), jnp.int32),
                       pltpu.VMEM((EB, d), jnp.float32),
                       pltpu.SemaphoreType.DMA],
    )
    def k(table_hbm, idx_hbm, out_hbm, idx_v, rows_v, sem):
        wid = lax.axis_index("s") * NC + lax.axis_index("c")

        def one(b):
            base = b * EB
            pltpu.sync_copy(idx_hbm.at[pl.ds(base, EB)], idx_v)
            pltpu.async_copy(table_hbm.at[idx_v], rows_v, sem).wait()
            pltpu.sync_copy(rows_v, out_hbm.at[pl.ds(base, EB)])

        nfull = nb // nw
        lax.fori_loop(0, nfull, lambda j, c: (one(j * nw + wid), c)[1], 0)

        @pl.when(nfull * nw + wid < nb)
        def _():
            one(nfull * nw + wid)

    return k(table, idx)


def _sc_segsum(e_arr, recv, zeros, n):
    e = recv.shape[0]
    nb = e // EB
    half = LAT // NC
    n_pad = ((n + NS * 8 - 1) // (NS * 8)) * (NS * 8)
    rows_per_tile = n_pad // NS
    mesh = plsc.VectorSubcoreMesh(core_axis_name="c", subcore_axis_name="s")

    @functools.partial(
        pl.kernel, mesh=mesh,
        out_type=jax.ShapeDtypeStruct((NC, n_pad, half), jnp.float32),
        scratch_types=[pltpu.VMEM((1, EB), jnp.int32),
                       pltpu.VMEM((EB, half), jnp.float32),
                       pltpu.VMEM_SHARED((n_pad, half), jnp.float32)],
    )
    def k(e_hbm, r_hbm, z_hbm, out_hbm, idx_v, ebuf, shared):
        c = lax.axis_index("c")
        s = lax.axis_index("s")
        pltpu.sync_copy(z_hbm,
                        shared.at[pl.ds(s * rows_per_tile, rows_per_tile)])
        plsc.subcore_barrier()

        def one(b):
            base = b * EB
            pltpu.sync_copy(r_hbm.at[pl.ds(base, EB)], idx_v.at[0])
            pltpu.sync_copy(e_hbm.at[pl.ds(base, EB), pl.ds(c * half, half)],
                            ebuf)
            pltpu.sync_copy(ebuf, shared.at[idx_v.at[0]], add=True)

        nfull = nb // NS
        lax.fori_loop(0, nfull, lambda j, cr: (one(j * NS + s), cr)[1], 0)

        @pl.when(nfull * NS + s < nb)
        def _():
            one(nfull * NS + s)

        plsc.subcore_barrier()
        pltpu.sync_copy(shared.at[pl.ds(s * rows_per_tile, rows_per_tile)],
                        out_hbm.at[c, pl.ds(s * rows_per_tile, rows_per_tile)])

    return k(e_arr, recv, zeros)


def kernel(nodes, edges, senders, receivers, n_node, params):
    n, f = nodes.shape
    e_cnt = edges.shape[0]

    sp = params['sender']
    x = _mlp2([(nodes, sp[0][0])], [], sp[0][1:], sp[1], block_rows=1000)
    x = x + (jnp.asarray(n_node) - n).astype(jnp.float32)

    ep = params['edge0']
    k3 = ep[0][0].shape[0]
    edges8 = jnp.concatenate(
        [edges, jnp.zeros((e_cnt, 8 - k3), jnp.float32)], axis=1)
    w0 = jnp.concatenate([ep[0][0], jnp.zeros((8 - k3, LAT), jnp.float32)],
                         axis=0)
    e = _mlp2([(edges8, w0)], [], ep[0][1:], ep[1], block_rows=640)

    n_pad = ((n + NS * 8 - 1) // (NS * 8)) * (NS * 8)
    zeros = jnp.zeros((n_pad // NS, LAT // NC), jnp.float32)

    for i in range(len(params['edge_steps'])):
        eps = params['edge_steps'][i]
        nps = params['node_steps'][i]
        w1 = eps[0][0]
        we, ws, wr = w1[:LAT], w1[LAT:2 * LAT], w1[2 * LAT:]
        ps, pr = _proj2(x, ws, wr, block_rows=1000)
        gs = _sc_gather(ps, senders)
        gr = _sc_gather(pr, receivers)
        e = _mlp2([(e, we)], [gs, gr], eps[0][1:], eps[1], block_rows=640)
        agg = _sc_segsum(e, receivers, zeros, n)
        wn1 = nps[0][0]
        half = LAT // NC
        x = _mlp2([(x, wn1[:LAT]),
                   (agg[0, :n], wn1[LAT:LAT + half]),
                   (agg[1, :n], wn1[LAT + half:])],
                  [], nps[0][1:], nps[1], block_rows=1000)

    return (x, e)


# speedup vs baseline: 1.8320x; 1.0955x over previous
import functools

import jax
import jax.numpy as jnp
from jax import lax
from jax.experimental import pallas as pl
from jax.experimental.pallas import tpu as pltpu
from jax.experimental.pallas import tpu_sc as plsc

NC, NS = 2, 16
EB = 128
LAT = 256


def _ln(h, g, be):
    mu = jnp.mean(h, axis=-1, keepdims=True)
    d = h - mu
    var = jnp.mean(d * d, axis=-1, keepdims=True)
    return d * lax.rsqrt(var + 1e-5) * g + be


def _mlp2_body(n_mm, n_add, *refs):
    xs = refs[:n_mm]
    ws = refs[n_mm:2 * n_mm]
    adds = refs[2 * n_mm:2 * n_mm + n_add]
    b1, g1, be1, w2, b2, g2, be2, out = refs[2 * n_mm + n_add:]
    acc = jnp.dot(xs[0][...], ws[0][...], preferred_element_type=jnp.float32)
    for k in range(1, n_mm):
        acc = acc + jnp.dot(xs[k][...], ws[k][...],
                            preferred_element_type=jnp.float32)
    for a in adds:
        acc = acc + a[...]
    h = _ln(jax.nn.relu(acc + b1[...]), g1[...], be1[...])
    h2 = jax.nn.relu(jnp.dot(h, w2[...], preferred_element_type=jnp.float32)
                     + b2[...])
    out[...] = _ln(h2, g2[...], be2[...])


def _mlp2(mm_inputs, add_inputs, l1, l2, block_rows):
    m = mm_inputs[0][0].shape[0]
    n_mm, n_add = len(mm_inputs), len(add_inputs)
    b1, g1, be1 = (p.reshape(1, LAT) for p in l1)
    w2 = l2[0]
    b2, g2, be2 = (p.reshape(1, LAT) for p in l2[1:])
    row_spec = lambda k: pl.BlockSpec((block_rows, k), lambda i: (i, 0))
    full_spec = lambda s: pl.BlockSpec(s, lambda i: (0, 0))
    in_specs = ([row_spec(x.shape[1]) for x, _ in mm_inputs]
                + [full_spec(w.shape) for _, w in mm_inputs]
                + [row_spec(LAT) for _ in add_inputs]
                + [full_spec((1, LAT))] * 3
                + [full_spec((LAT, LAT))]
                + [full_spec((1, LAT))] * 3)
    args = ([x for x, _ in mm_inputs] + [w for _, w in mm_inputs]
            + list(add_inputs) + [b1, g1, be1, w2, b2, g2, be2])
    return pl.pallas_call(
        functools.partial(_mlp2_body, n_mm, n_add),
        grid=(m // block_rows,),
        in_specs=in_specs,
        out_specs=pl.BlockSpec((block_rows, LAT), lambda i: (i, 0)),
        out_shape=jax.ShapeDtypeStruct((m, LAT), jnp.float32),
    )(*args)


def _proj_body(x_ref, wa_ref, wb_ref, oa_ref, ob_ref):
    x = x_ref[...]
    oa_ref[...] = jnp.dot(x, wa_ref[...], preferred_element_type=jnp.float32)
    ob_ref[...] = jnp.dot(x, wb_ref[...], preferred_element_type=jnp.float32)


def _proj2(x, wa, wb, block_rows):
    m = x.shape[0]
    return pl.pallas_call(
        _proj_body,
        grid=(m // block_rows,),
        in_specs=[pl.BlockSpec((block_rows, LAT), lambda i: (i, 0)),
                  pl.BlockSpec((LAT, LAT), lambda i: (0, 0)),
                  pl.BlockSpec((LAT, LAT), lambda i: (0, 0))],
        out_specs=[pl.BlockSpec((block_rows, LAT), lambda i: (i, 0)),
                   pl.BlockSpec((block_rows, LAT), lambda i: (i, 0))],
        out_shape=[jax.ShapeDtypeStruct((m, LAT), jnp.float32),
                   jax.ShapeDtypeStruct((m, LAT), jnp.float32)],
    )(x, wa, wb)


def _sc_gather(table, idx):
    e, d = idx.shape[0], table.shape[1]
    nb = e // EB
    nw = NC * NS
    mesh = plsc.VectorSubcoreMesh(core_axis_name="c", subcore_axis_name="s")

    @functools.partial(
        pl.kernel, mesh=mesh,
        out_type=jax.ShapeDtypeStruct((e, d), jnp.float32),
        scratch_types=[pltpu.VMEM((EB,), jnp.int32),
                       pltpu.VMEM((EB, d), jnp.float32),
                       pltpu.SemaphoreType.DMA],
    )
    def k(table_hbm, idx_hbm, out_hbm, idx_v, rows_v, sem):
        wid = lax.axis_index("s") * NC + lax.axis_index("c")

        def one(b):
            base = b * EB
            pltpu.sync_copy(idx_hbm.at[pl.ds(base, EB)], idx_v)
            pltpu.async_copy(table_hbm.at[idx_v], rows_v, sem).wait()
            pltpu.sync_copy(rows_v, out_hbm.at[pl.ds(base, EB)])

        nfull = nb // nw
        lax.fori_loop(0, nfull, lambda j, c: (one(j * nw + wid), c)[1], 0)

        @pl.when(nfull * nw + wid < nb)
        def _():
            one(nfull * nw + wid)

    return k(table, idx)


def _sc_segsum(e_arr, recv, zeros, n):
    e = recv.shape[0]
    nb = e // EB
    half = LAT // NC
    n_pad = ((n + NS * 8 - 1) // (NS * 8)) * (NS * 8)
    rows_per_tile = n_pad // NS
    mesh = plsc.VectorSubcoreMesh(core_axis_name="c", subcore_axis_name="s")

    @functools.partial(
        pl.kernel, mesh=mesh,
        out_type=jax.ShapeDtypeStruct((NC, n_pad, half), jnp.float32),
        scratch_types=[pltpu.VMEM((2, EB), jnp.int32),
                       pltpu.VMEM((EB, half), jnp.float32),
                       pltpu.VMEM((EB, half), jnp.float32),
                       pltpu.VMEM_SHARED((n_pad, half), jnp.float32),
                       pltpu.SemaphoreType.DMA, pltpu.SemaphoreType.DMA],
    )
    def k(e_hbm, r_hbm, z_hbm, out_hbm, idx_v, eb0, eb1, shared, se0, se1):
        c = lax.axis_index("c")
        s = lax.axis_index("s")
        ebuf = (eb0, eb1)
        se = (se0, se1)
        pltpu.sync_copy(z_hbm,
                        shared.at[pl.ds(s * rows_per_tile, rows_per_tile)])
        plsc.subcore_barrier()

        def start_e(b, p):
            base = b * EB
            pltpu.sync_copy(r_hbm.at[pl.ds(base, EB)], idx_v.at[p])
            pltpu.async_copy(
                e_hbm.at[pl.ds(base, EB), pl.ds(c * half, half)],
                ebuf[p], se[p])

        def wait_add(p):
            pltpu.make_async_copy(
                e_hbm.at[pl.ds(0, EB), pl.ds(0, half)], ebuf[p],
                se[p]).wait()
            pltpu.sync_copy(ebuf[p], shared.at[idx_v.at[p]], add=True)

        def one(b):
            start_e(b, 0)
            wait_add(0)

        nfull = nb // NS
        start_e(s, 0)

        def pair(j, cr):
            b0 = (2 * j) * NS + s
            b1 = (2 * j + 1) * NS + s
            start_e(b1, 1)
            wait_add(0)

            @pl.when(2 * j + 2 < nfull)
            def _():
                start_e(b0 + 2 * NS, 0)

            wait_add(1)
            return cr

        lax.fori_loop(0, nfull // 2, pair, 0)

        @pl.when(nfull * NS + s < nb)
        def _():
            one(nfull * NS + s)

        plsc.subcore_barrier()
        pltpu.sync_copy(shared.at[pl.ds(s * rows_per_tile, rows_per_tile)],
                        out_hbm.at[c, pl.ds(s * rows_per_tile, rows_per_tile)])

    return k(e_arr, recv, zeros)


def kernel(nodes, edges, senders, receivers, n_node, params):
    n, f = nodes.shape
    e_cnt = edges.shape[0]

    sp = params['sender']
    x = _mlp2([(nodes, sp[0][0])], [], sp[0][1:], sp[1], block_rows=1000)
    x = x + (jnp.asarray(n_node) - n).astype(jnp.float32)

    ep = params['edge0']
    k3 = ep[0][0].shape[0]
    edges8 = jnp.concatenate(
        [edges, jnp.zeros((e_cnt, 8 - k3), jnp.float32)], axis=1)
    w0 = jnp.concatenate([ep[0][0], jnp.zeros((8 - k3, LAT), jnp.float32)],
                         axis=0)
    e = _mlp2([(edges8, w0)], [], ep[0][1:], ep[1], block_rows=640)

    n_pad = ((n + NS * 8 - 1) // (NS * 8)) * (NS * 8)
    zeros = jnp.zeros((n_pad // NS, LAT // NC), jnp.float32)

    for i in range(len(params['edge_steps'])):
        eps = params['edge_steps'][i]
        nps = params['node_steps'][i]
        w1 = eps[0][0]
        we, ws, wr = w1[:LAT], w1[LAT:2 * LAT], w1[2 * LAT:]
        ps, pr = _proj2(x, ws, wr, block_rows=1000)
        gs = _sc_gather(ps, senders)
        gr = _sc_gather(pr, receivers)
        e = _mlp2([(e, we)], [gs, gr], eps[0][1:], eps[1], block_rows=640)
        agg = _sc_segsum(e, receivers, zeros, n)
        wn1 = nps[0][0]
        half = LAT // NC
        x = _mlp2([(x, wn1[:LAT]),
                   (agg[0, :n], wn1[LAT:LAT + half]),
                   (agg[1, :n], wn1[LAT + half:])],
                  [], nps[0][1:], nps[1], block_rows=1000)

    return (x, e)


# speedup vs baseline: 1.9210x; 1.0486x over previous
import functools

import jax
import jax.numpy as jnp
from jax import lax
from jax.experimental import pallas as pl
from jax.experimental.pallas import tpu as pltpu
from jax.experimental.pallas import tpu_sc as plsc

NC, NS = 2, 16
EB = 128
LAT = 256


def _ln(h, g, be):
    mu = jnp.mean(h, axis=-1, keepdims=True)
    d = h - mu
    var = jnp.mean(d * d, axis=-1, keepdims=True)
    return d * lax.rsqrt(var + 1e-5) * g + be


def _mlp2_body(n_mm, n_add, *refs):
    xs = refs[:n_mm]
    ws = refs[n_mm:2 * n_mm]
    adds = refs[2 * n_mm:2 * n_mm + n_add]
    b1, g1, be1, w2, b2, g2, be2, out = refs[2 * n_mm + n_add:]
    acc = jnp.dot(xs[0][...], ws[0][...], preferred_element_type=jnp.float32)
    for k in range(1, n_mm):
        acc = acc + jnp.dot(xs[k][...], ws[k][...],
                            preferred_element_type=jnp.float32)
    for a in adds:
        acc = acc + a[...]
    h = _ln(jax.nn.relu(acc + b1[...]), g1[...], be1[...])
    h2 = jax.nn.relu(jnp.dot(h, w2[...], preferred_element_type=jnp.float32)
                     + b2[...])
    out[...] = _ln(h2, g2[...], be2[...])


def _mlp2(mm_inputs, add_inputs, l1, l2, block_rows):
    m = mm_inputs[0][0].shape[0]
    n_mm, n_add = len(mm_inputs), len(add_inputs)
    b1, g1, be1 = (p.reshape(1, LAT) for p in l1)
    w2 = l2[0]
    b2, g2, be2 = (p.reshape(1, LAT) for p in l2[1:])
    row_spec = lambda k: pl.BlockSpec((block_rows, k), lambda i: (i, 0))
    full_spec = lambda s: pl.BlockSpec(s, lambda i: (0, 0))
    in_specs = ([row_spec(x.shape[1]) for x, _ in mm_inputs]
                + [full_spec(w.shape) for _, w in mm_inputs]
                + [row_spec(LAT) for _ in add_inputs]
                + [full_spec((1, LAT))] * 3
                + [full_spec((LAT, LAT))]
                + [full_spec((1, LAT))] * 3)
    args = ([x for x, _ in mm_inputs] + [w for _, w in mm_inputs]
            + list(add_inputs) + [b1, g1, be1, w2, b2, g2, be2])
    return pl.pallas_call(
        functools.partial(_mlp2_body, n_mm, n_add),
        grid=(m // block_rows,),
        in_specs=in_specs,
        out_specs=pl.BlockSpec((block_rows, LAT), lambda i: (i, 0)),
        out_shape=jax.ShapeDtypeStruct((m, LAT), jnp.float32),
    )(*args)


def _proj_body(x_ref, wa_ref, wb_ref, oa_ref, ob_ref):
    x = x_ref[...]
    oa_ref[...] = jnp.dot(x, wa_ref[...], preferred_element_type=jnp.float32)
    ob_ref[...] = jnp.dot(x, wb_ref[...], preferred_element_type=jnp.float32)


def _proj2(x, wa, wb, block_rows):
    m = x.shape[0]
    return pl.pallas_call(
        _proj_body,
        grid=(m // block_rows,),
        in_specs=[pl.BlockSpec((block_rows, LAT), lambda i: (i, 0)),
                  pl.BlockSpec((LAT, LAT), lambda i: (0, 0)),
                  pl.BlockSpec((LAT, LAT), lambda i: (0, 0))],
        out_specs=[pl.BlockSpec((block_rows, LAT), lambda i: (i, 0)),
                   pl.BlockSpec((block_rows, LAT), lambda i: (i, 0))],
        out_shape=[jax.ShapeDtypeStruct((m, LAT), jnp.float32),
                   jax.ShapeDtypeStruct((m, LAT), jnp.float32)],
    )(x, wa, wb)


def _sc_gather(table, idx):
    e, d = idx.shape[0], table.shape[1]
    nb = e // EB
    nw = NC * NS
    mesh = plsc.VectorSubcoreMesh(core_axis_name="c", subcore_axis_name="s")

    @functools.partial(
        pl.kernel, mesh=mesh,
        out_type=jax.ShapeDtypeStruct((e, d), jnp.float32),
        scratch_types=[pltpu.VMEM((EB,), jnp.int32),
                       pltpu.VMEM((EB,), jnp.int32),
                       pltpu.VMEM((EB, d), jnp.float32),
                       pltpu.VMEM((EB, d), jnp.float32),
                       pltpu.SemaphoreType.DMA, pltpu.SemaphoreType.DMA],
    )
    def k(table_hbm, idx_hbm, out_hbm, i0, i1, r0, r1, sg0, sg1):
        wid = lax.axis_index("s") * NC + lax.axis_index("c")
        ibuf = (i0, i1)
        rows = (r0, r1)
        sg = (sg0, sg1)

        def start(b, p):
            base = b * EB
            pltpu.sync_copy(idx_hbm.at[pl.ds(base, EB)], ibuf[p])
            pltpu.async_copy(table_hbm.at[ibuf[p]], rows[p], sg[p])

        def finish(b, p):
            pltpu.make_async_copy(table_hbm.at[ibuf[p]], rows[p],
                                  sg[p]).wait()
            pltpu.sync_copy(rows[p], out_hbm.at[pl.ds(b * EB, EB)])

        def one(b):
            start(b, 0)
            finish(b, 0)

        nfull = nb // nw
        npair = nfull // 2

        def gb(k2):
            return k2 * nw + wid

        start(gb(0), 0)

        def pair(j, cr):
            k0 = 2 * j
            k1 = k0 + 1
            start(gb(k1), 1)
            finish(gb(k0), 0)

            @pl.when(k0 + 2 < 2 * npair)
            def _():
                start(gb(k0 + 2), 0)

            finish(gb(k1), 1)
            return cr

        lax.fori_loop(0, npair, pair, 0)

        @pl.when(2 * npair < nfull)
        def _():
            one(gb(2 * npair))

        @pl.when(nfull * nw + wid < nb)
        def _():
            one(nfull * nw + wid)

    return k(table, idx)


def _sc_segsum(e_arr, recv, zeros, n):
    e = recv.shape[0]
    nb = e // EB
    half = LAT // NC
    n_pad = ((n + NS * 8 - 1) // (NS * 8)) * (NS * 8)
    rows_per_tile = n_pad // NS
    mesh = plsc.VectorSubcoreMesh(core_axis_name="c", subcore_axis_name="s")

    @functools.partial(
        pl.kernel, mesh=mesh,
        out_type=jax.ShapeDtypeStruct((NC, n_pad, half), jnp.float32),
        scratch_types=[pltpu.VMEM((2, EB), jnp.int32),
                       pltpu.VMEM((EB, half), jnp.float32),
                       pltpu.VMEM((EB, half), jnp.float32),
                       pltpu.VMEM_SHARED((n_pad, half), jnp.float32),
                       pltpu.SemaphoreType.DMA, pltpu.SemaphoreType.DMA],
    )
    def k(e_hbm, r_hbm, z_hbm, out_hbm, idx_v, eb0, eb1, shared, se0, se1):
        c = lax.axis_index("c")
        s = lax.axis_index("s")
        ebuf = (eb0, eb1)
        se = (se0, se1)
        pltpu.sync_copy(z_hbm,
                        shared.at[pl.ds(s * rows_per_tile, rows_per_tile)])
        plsc.subcore_barrier()

        def start_e(b, p):
            base = b * EB
            pltpu.sync_copy(r_hbm.at[pl.ds(base, EB)], idx_v.at[p])
            pltpu.async_copy(
                e_hbm.at[pl.ds(base, EB), pl.ds(c * half, half)],
                ebuf[p], se[p])

        def wait_add(p):
            pltpu.make_async_copy(
                e_hbm.at[pl.ds(0, EB), pl.ds(0, half)], ebuf[p],
                se[p]).wait()
            pltpu.sync_copy(ebuf[p], shared.at[idx_v.at[p]], add=True)

        def one(b):
            start_e(b, 0)
            wait_add(0)

        nfull = nb // NS
        start_e(s, 0)

        def pair(j, cr):
            b0 = (2 * j) * NS + s
            b1 = (2 * j + 1) * NS + s
            start_e(b1, 1)
            wait_add(0)

            @pl.when(2 * j + 2 < nfull)
            def _():
                start_e(b0 + 2 * NS, 0)

            wait_add(1)
            return cr

        lax.fori_loop(0, nfull // 2, pair, 0)

        @pl.when(nfull * NS + s < nb)
        def _():
            one(nfull * NS + s)

        plsc.subcore_barrier()
        pltpu.sync_copy(shared.at[pl.ds(s * rows_per_tile, rows_per_tile)],
                        out_hbm.at[c, pl.ds(s * rows_per_tile, rows_per_tile)])

    return k(e_arr, recv, zeros)


def kernel(nodes, edges, senders, receivers, n_node, params):
    n, f = nodes.shape
    e_cnt = edges.shape[0]

    sp = params['sender']
    x = _mlp2([(nodes, sp[0][0])], [], sp[0][1:], sp[1], block_rows=1000)
    x = x + (jnp.asarray(n_node) - n).astype(jnp.float32)

    ep = params['edge0']
    k3 = ep[0][0].shape[0]
    edges8 = jnp.concatenate(
        [edges, jnp.zeros((e_cnt, 8 - k3), jnp.float32)], axis=1)
    w0 = jnp.concatenate([ep[0][0], jnp.zeros((8 - k3, LAT), jnp.float32)],
                         axis=0)
    e = _mlp2([(edges8, w0)], [], ep[0][1:], ep[1], block_rows=640)

    n_pad = ((n + NS * 8 - 1) // (NS * 8)) * (NS * 8)
    zeros = jnp.zeros((n_pad // NS, LAT // NC), jnp.float32)

    for i in range(len(params['edge_steps'])):
        eps = params['edge_steps'][i]
        nps = params['node_steps'][i]
        w1 = eps[0][0]
        we, ws, wr = w1[:LAT], w1[LAT:2 * LAT], w1[2 * LAT:]
        ps, pr = _proj2(x, ws, wr, block_rows=1000)
        gs = _sc_gather(ps, senders)
        gr = _sc_gather(pr, receivers)
        e = _mlp2([(e, we)], [gs, gr], eps[0][1:], eps[1], block_rows=640)
        agg = _sc_segsum(e, receivers, zeros, n)
        wn1 = nps[0][0]
        half = LAT // NC
        x = _mlp2([(x, wn1[:LAT]),
                   (agg[0, :n], wn1[LAT:LAT + half]),
                   (agg[1, :n], wn1[LAT + half:])],
                  [], nps[0][1:], nps[1], block_rows=1000)

    return (x, e)
